# baseline (speedup 1.0000x reference)
import functools

import jax
import jax.numpy as jnp
from jax.experimental import pallas as pl
from jax.experimental.pallas import tpu as pltpu


def _fused_kernel(scale_ref, x_ref, wt_ref, proto_ref, out_ref):
    t = jnp.dot(x_ref[...], wt_ref[...], preferred_element_type=jnp.float32)
    z = jnp.tanh(t)
    nrm = jnp.maximum(
        jnp.sqrt(jnp.sum(z * z, axis=-1, keepdims=True)), 1e-6
    )
    z = z / nrm
    p = proto_ref[...]
    pn = p / jnp.maximum(
        jnp.sqrt(jnp.sum(p * p, axis=-1, keepdims=True)), 1e-12
    )
    scale = 3.0 * scale_ref[0]
    logits = scale * jnp.dot(z, pn.T, preferred_element_type=jnp.float32)
    m = jnp.max(logits, axis=-1, keepdims=True)
    e = jnp.exp(logits - m)
    out_ref[...] = e / jnp.sum(e, axis=-1, keepdims=True)


@functools.partial(jax.jit, static_argnames=("block_m",))
def _run(x2d, wt, prototypes, hamming_scale, block_m):
    n_rows = x2d.shape[0]
    grid = (n_rows // block_m,)
    return pl.pallas_call(
        _fused_kernel,
        grid_spec=pltpu.PrefetchScalarGridSpec(
            num_scalar_prefetch=1,
            grid=grid,
            in_specs=[
                pl.BlockSpec((block_m, x2d.shape[1]), lambda i, s: (i, 0)),
                pl.BlockSpec((x2d.shape[1], wt.shape[1]), lambda i, s: (0, 0)),
                pl.BlockSpec(prototypes.shape, lambda i, s: (0, 0)),
            ],
            out_specs=pl.BlockSpec((block_m, 8), lambda i, s: (i, 0)),
        ),
        out_shape=jax.ShapeDtypeStruct((n_rows, 8), jnp.float32),
    )(hamming_scale.reshape(1), x2d, wt, prototypes)


def kernel(x, W, prototypes, hamming_scale):
    b, s, d = x.shape
    x2d = x.reshape(b * s, d)
    out = _run(x2d, W.T, prototypes, jnp.asarray(hamming_scale, jnp.float32),
               block_m=1024)
    return out.reshape(b, s, prototypes.shape[0])


# speedup vs baseline: 1.0741x; 1.0741x over previous
import functools

import jax
import jax.numpy as jnp
from jax.experimental import pallas as pl
from jax.experimental.pallas import tpu as pltpu


def _fused_kernel(x_ref, wt_ref, pnt_ref, out_ref):
    t = jnp.dot(x_ref[...], wt_ref[...], preferred_element_type=jnp.float32)
    u = jnp.tanh(t)
    s = jnp.sum(u * u, axis=-1, keepdims=True)
    r = jax.lax.rsqrt(jnp.maximum(s, 1e-12))
    d = jnp.dot(u, pnt_ref[...], preferred_element_type=jnp.float32)
    e = jnp.exp(d * r)
    out_ref[...] = e / jnp.sum(e, axis=-1, keepdims=True)


@functools.partial(jax.jit, static_argnames=("block_m",))
def _run(x2d, wt, pnt, block_m):
    n_rows, d = x2d.shape
    grid = (n_rows // block_m,)
    return pl.pallas_call(
        _fused_kernel,
        grid=grid,
        in_specs=[
            pl.BlockSpec((block_m, d), lambda i: (i, 0)),
            pl.BlockSpec(wt.shape, lambda i: (0, 0)),
            pl.BlockSpec(pnt.shape, lambda i: (0, 0)),
        ],
        out_specs=pl.BlockSpec((block_m, 8), lambda i: (i, 0)),
        out_shape=jax.ShapeDtypeStruct((n_rows, 8), jnp.float32),
        compiler_params=pltpu.CompilerParams(
            dimension_semantics=("arbitrary",),
        ),
    )(x2d, wt, pnt)


def kernel(x, W, prototypes, hamming_scale):
    b, s, d = x.shape
    x2d = x.reshape(b * s, d)
    pn = prototypes / jnp.maximum(
        jnp.linalg.norm(prototypes, axis=-1, keepdims=True), 1e-12
    )
    pnt = (3.0 * jnp.asarray(hamming_scale, jnp.float32)) * pn.T
    out = _run(x2d, W.T, pnt, block_m=2048)
    return out.reshape(b, s, prototypes.shape[0])
